# baseline (speedup 1.0000x reference)
import jax
import jax.numpy as jnp
from jax.experimental import pallas as pl
from jax.experimental.pallas import tpu as pltpu

HEADS = 2
HID = 32
T = 12
T_OUT = 12
N = 512
NUM = 8192
NL = NUM - N


def _leaky(x):
    return jnp.maximum(x, 0.2 * x)


def _gates(g):
    ii = jax.nn.sigmoid(g[:, 0:HID])
    ff = jax.nn.sigmoid(g[:, HID:2 * HID])
    gg = jnp.tanh(g[:, 2 * HID:3 * HID])
    oo = jax.nn.sigmoid(g[:, 3 * HID:4 * HID])
    return ii, ff, gg, oo


def _fused_kernel(x24_ref, x24t_ref, at_ref, wgat_ref, psrcT_ref, psrc_ref,
                  pdst_ref, bg_ref, wl1a_ref, wl1l_ref, b0_ref, b0l_ref,
                  wl2_ref, b1_ref, wfc_ref, bfc_ref, out_ref):
    xb = x24_ref[...]
    xb0 = xb[0:N, :]
    xbl = xb[N:, :]
    at = at_ref[...]
    wg = wgat_ref[...]
    bg = bg_ref[...]
    asr = jnp.dot(psrcT_ref[...], x24t_ref[...],
                  preferred_element_type=jnp.float32)
    asc = jnp.dot(xb0, psrc_ref[...], preferred_element_type=jnp.float32)
    adc = jnp.dot(xb0, pdst_ref[...], preferred_element_type=jnp.float32)

    wl1a = wl1a_ref[...]
    wl1l = wl1l_ref[...]
    b0 = b0_ref[...]
    b0l = b0l_ref[...]
    wl2 = wl2_ref[...]
    b1 = b1_ref[...]
    za = jnp.zeros((N, HID), jnp.float32)
    zl = jnp.zeros((NL, HID), jnp.float32)
    h1a, c1a, h2a, c2a = za, za, za, za
    h1l, c1l, h2l, c2l = zl, zl, zl, zl
    for t in range(T):
        ht = jnp.dot(xb0[:, 2 * t:2 * t + 2], wg,
                     preferred_element_type=jnp.float32)
        acc = None
        for h in range(HEADS):
            c = 2 * t + h
            ad = adc[:, c:c + 1]
            dv = _leaky(ad + asc[:, c:c + 1])
            Em = jnp.exp(_leaky(ad + asr[c:c + 1, :]) - dv) * at
            denom = jnp.sum(Em, axis=1, keepdims=True) + (1.0 + 1e-16)
            hh = ht[:, HID * h:HID * h + HID]
            agg = (jnp.dot(Em, hh, preferred_element_type=jnp.float32)
                   + hh) / denom
            acc = agg if acc is None else acc + agg
        xattn = 0.5 * acc + bg
        ga = (jnp.dot(jax.lax.concatenate([xattn, h1a], 1), wl1a,
                      preferred_element_type=jnp.float32) + b0)
        gl = (jnp.dot(jax.lax.concatenate([xbl[:, 2 * t:2 * t + 2], h1l], 1),
                      wl1l, preferred_element_type=jnp.float32) + b0l)
        ii, ff, gg, oo = _gates(ga)
        c1a = ff * c1a + ii * gg
        h1a = oo * jnp.tanh(c1a)
        ii, ff, gg, oo = _gates(gl)
        c1l = ff * c1l + ii * gg
        h1l = oo * jnp.tanh(c1l)
        ga = (jnp.dot(jax.lax.concatenate([h1a, h2a], 1), wl2,
                      preferred_element_type=jnp.float32) + b1)
        gl = (jnp.dot(jax.lax.concatenate([h1l, h2l], 1), wl2,
                      preferred_element_type=jnp.float32) + b1)
        ii, ff, gg, oo = _gates(ga)
        c2a = ff * c2a + ii * gg
        h2a = oo * jnp.tanh(c2a)
        ii, ff, gg, oo = _gates(gl)
        c2l = ff * c2l + ii * gg
        h2l = oo * jnp.tanh(c2l)
    wfc = wfc_ref[...]
    bfc = bfc_ref[...]
    out_ref[0:N, :] = (jnp.dot(h2a, wfc,
                               preferred_element_type=jnp.float32) + bfc)
    out_ref[N:, :] = (jnp.dot(h2l, wfc,
                              preferred_element_type=jnp.float32) + bfc)


def kernel(A_hat, X, W_gat, att_src, att_dst, b_gat, W_ih0, W_hh0, b_ih0,
           b_hh0, W_ih1, W_hh1, b_ih1, b_hh1, W_fc, b_fc):
    B, n, t, F = X.shape
    num = B * n
    x24 = X.reshape(num, t * F)
    x24t = x24[:n].T
    atT = (A_hat.T != 0).astype(jnp.float32)

    p_src = jnp.stack(
        [W_gat[:, h * HID:(h + 1) * HID] @ att_src[0, h] for h in range(HEADS)],
        axis=1)
    p_dst = jnp.stack(
        [W_gat[:, h * HID:(h + 1) * HID] @ att_dst[0, h] for h in range(HEADS)],
        axis=1)
    eyeT = jnp.eye(t, dtype=jnp.float32)
    Psrc = jnp.kron(eyeT, p_src)
    Pdst = jnp.kron(eyeT, p_dst)
    Wcomb = 0.5 * (W_gat[:, :HID] + W_gat[:, HID:])
    bg = b_gat[None, :]

    wi0 = W_ih0.T
    wh0 = W_hh0.T
    wl1a = jnp.concatenate([wi0, wh0], axis=0)
    wl1l = jnp.concatenate([Wcomb @ wi0, wh0], axis=0)
    b0 = (b_ih0 + b_hh0)[None, :]
    b0l = b0 + (b_gat @ wi0)[None, :]
    wl2 = jnp.concatenate([W_ih1.T, W_hh1.T], axis=0)
    b1 = (b_ih1 + b_hh1)[None, :]
    wfc = W_fc.T
    bfc = b_fc[None, :]

    out24 = pl.pallas_call(
        _fused_kernel,
        out_shape=jax.ShapeDtypeStruct((num, T_OUT * F), jnp.float32),
    )(x24, x24t, atT, W_gat, Psrc.T, Psrc, Pdst, bg,
      wl1a, wl1l, b0, b0l, wl2, b1, wfc, bfc)
    return out24.reshape(B, n, T_OUT, F)


# speedup vs baseline: 1.0951x; 1.0951x over previous
import jax
import jax.numpy as jnp
from jax.experimental import pallas as pl
from jax.experimental.pallas import tpu as pltpu

HEADS = 2
HID = 32
T = 12
T_OUT = 12
N = 512
NUM = 8192


def _leaky(x):
    return jnp.maximum(x, 0.2 * x)


def _fused_kernel(x24_ref, x24t_ref, at_ref, wgat_ref, psrcT_ref, psrc_ref,
                  pdst_ref, wcomb_ref, bg_ref, wi0_ref, wh0_ref, b0_ref,
                  wi1_ref, wh1_ref, b1_ref, wfc_ref, bfc_ref, out_ref):
    xb = x24_ref[...]
    xb0 = xb[0:N, :]
    at = at_ref[...]
    wg = wgat_ref[...]
    bg = bg_ref[...]
    wc = wcomb_ref[...]
    asr = jnp.dot(psrcT_ref[...], x24t_ref[...],
                  preferred_element_type=jnp.float32)
    asr2 = 0.2 * asr
    asc = jnp.dot(xb0, psrc_ref[...], preferred_element_type=jnp.float32)
    adc = jnp.dot(xb0, pdst_ref[...], preferred_element_type=jnp.float32)

    def attn(t):
        ht = jnp.dot(xb0[:, 2 * t:2 * t + 2], wg,
                     preferred_element_type=jnp.float32)
        acc = None
        for h in range(HEADS):
            c = 2 * t + h
            ad = adc[:, c:c + 1]
            dv = _leaky(ad + asc[:, c:c + 1])
            u = (ad - dv) + asr[c:c + 1, :]
            v = (0.2 * ad - dv) + asr2[c:c + 1, :]
            Em = jnp.exp(jnp.maximum(u, v)) * at
            denom = jnp.sum(Em, axis=1, keepdims=True) + (1.0 + 1e-16)
            hh = ht[:, HID * h:HID * h + HID]
            agg = (jnp.dot(Em, hh, preferred_element_type=jnp.float32)
                   + hh) / denom
            acc = agg if acc is None else acc + agg
        return 0.5 * acc + bg

    wi0 = wi0_ref[...]
    wh0 = wh0_ref[...]
    b0 = b0_ref[...]
    wi1 = wi1_ref[...]
    wh1 = wh1_ref[...]
    b1 = b1_ref[...]
    z = jnp.zeros((NUM, HID), jnp.float32)
    h1, c1, h2, c2 = z, z, z, z
    xattn_next = attn(0)
    for t in range(T):
        xattn = xattn_next
        if t + 1 < T:
            xattn_next = attn(t + 1)
        xlin = (jnp.dot(xb[N:, 2 * t:2 * t + 2], wc,
                        preferred_element_type=jnp.float32) + bg)
        xt = jax.lax.concatenate([xattn, xlin], 0)
        g = (jnp.dot(xt, wi0, preferred_element_type=jnp.float32)
             + jnp.dot(h1, wh0, preferred_element_type=jnp.float32) + b0)
        ii = jax.nn.sigmoid(g[:, 0:HID])
        ff = jax.nn.sigmoid(g[:, HID:2 * HID])
        gg = jnp.tanh(g[:, 2 * HID:3 * HID])
        oo = jax.nn.sigmoid(g[:, 3 * HID:4 * HID])
        c1 = ff * c1 + ii * gg
        h1 = oo * jnp.tanh(c1)
        g = (jnp.dot(h1, wi1, preferred_element_type=jnp.float32)
             + jnp.dot(h2, wh1, preferred_element_type=jnp.float32) + b1)
        ii = jax.nn.sigmoid(g[:, 0:HID])
        ff = jax.nn.sigmoid(g[:, HID:2 * HID])
        gg = jnp.tanh(g[:, 2 * HID:3 * HID])
        oo = jax.nn.sigmoid(g[:, 3 * HID:4 * HID])
        c2 = ff * c2 + ii * gg
        h2 = oo * jnp.tanh(c2)
    out_ref[...] = (jnp.dot(h2, wfc_ref[...],
                            preferred_element_type=jnp.float32) + bfc_ref[...])


def kernel(A_hat, X, W_gat, att_src, att_dst, b_gat, W_ih0, W_hh0, b_ih0,
           b_hh0, W_ih1, W_hh1, b_ih1, b_hh1, W_fc, b_fc):
    B, n, t, F = X.shape
    num = B * n
    x24 = X.reshape(num, t * F)
    x24t = x24[:n].T
    atT = (A_hat.T != 0).astype(jnp.float32)

    p_src = jnp.stack(
        [W_gat[:, h * HID:(h + 1) * HID] @ att_src[0, h] for h in range(HEADS)],
        axis=1)
    p_dst = jnp.stack(
        [W_gat[:, h * HID:(h + 1) * HID] @ att_dst[0, h] for h in range(HEADS)],
        axis=1)
    eyeT = jnp.eye(t, dtype=jnp.float32)
    Psrc = jnp.kron(eyeT, p_src)
    Pdst = jnp.kron(eyeT, p_dst)
    Wcomb = 0.5 * (W_gat[:, :HID] + W_gat[:, HID:])
    bg = b_gat[None, :]

    wi0 = W_ih0.T
    wh0 = W_hh0.T
    b0 = (b_ih0 + b_hh0)[None, :]
    wi1 = W_ih1.T
    wh1 = W_hh1.T
    b1 = (b_ih1 + b_hh1)[None, :]
    wfc = W_fc.T
    bfc = b_fc[None, :]

    out24 = pl.pallas_call(
        _fused_kernel,
        out_shape=jax.ShapeDtypeStruct((num, T_OUT * F), jnp.float32),
    )(x24, x24t, atT, W_gat, Psrc.T, Psrc, Pdst, Wcomb, bg,
      wi0, wh0, b0, wi1, wh1, b1, wfc, bfc)
    return out24.reshape(B, n, T_OUT, F)
